# baseline (speedup 1.0000x reference)
import jax
import jax.numpy as jnp
from jax.experimental import pallas as pl


def _moe_kernel(x_ref, wg_ref, bg_ref, w1_ref, b1_ref, w2_ref, b2_ref, o_ref):
    xb = x_ref[...]
    g = jnp.dot(xb, wg_ref[...], preferred_element_type=jnp.float32)
    g = g + bg_ref[...]
    m = jnp.max(g, axis=-1, keepdims=True)
    p = 1.0 / jnp.sum(jnp.exp(g - m), axis=-1, keepdims=True)
    xb16 = xb.astype(jnp.bfloat16)
    eh = w1_ref.shape[1]
    n_chunks = 4
    ck = eh // n_chunks
    out = jnp.sum(b2_ref[...], axis=0, keepdims=True)
    for k in range(n_chunks):
        sl = pl.ds(k * ck, ck)
        h = jnp.dot(xb16, w1_ref[:, sl], preferred_element_type=jnp.float32)
        h = h + b1_ref[:, sl]
        h = 0.5 * h * (1.0 + jax.lax.erf(h * 0.7071067811865476))
        out = out + jnp.dot(h.astype(jnp.bfloat16), w2_ref[sl, :],
                            preferred_element_type=jnp.float32)
    o_ref[...] = out * p


def kernel(x, Wg, bg, W1, b1, W2, b2):
    B, S, D = x.shape
    E, _, H = W1.shape
    EH = E * H
    M = B * S
    TM = 1024

    xf = x.reshape(M, D)
    W1c = W1.transpose(1, 0, 2).reshape(D, EH).astype(jnp.bfloat16)
    b1c = b1.reshape(1, EH)
    W2c = W2.reshape(EH, D).astype(jnp.bfloat16)
    bg2 = bg.reshape(1, E)

    out = pl.pallas_call(
        _moe_kernel,
        grid=(M // TM,),
        in_specs=[
            pl.BlockSpec((TM, D), lambda i: (i, 0)),
            pl.BlockSpec((D, E), lambda i: (0, 0)),
            pl.BlockSpec((1, E), lambda i: (0, 0)),
            pl.BlockSpec((D, EH), lambda i: (0, 0)),
            pl.BlockSpec((1, EH), lambda i: (0, 0)),
            pl.BlockSpec((EH, D), lambda i: (0, 0)),
            pl.BlockSpec((E, D), lambda i: (0, 0)),
        ],
        out_specs=pl.BlockSpec((TM, D), lambda i: (i, 0)),
        out_shape=jax.ShapeDtypeStruct((M, D), jnp.float32),
    )(xf, Wg, bg2, W1c, b1c, W2c, b2)
    return out.reshape(B, S, D)


# speedup vs baseline: 1.0633x; 1.0633x over previous
import jax
import jax.numpy as jnp
from jax.experimental import pallas as pl


def _moe_kernel(x_ref, w1g_ref, b1g_ref, w2_ref, b2_ref, o_ref):
    eh = w2_ref.shape[0]
    e = b2_ref.shape[0]
    xb16 = x_ref[...].astype(jnp.bfloat16)
    hg = jnp.dot(xb16, w1g_ref[...], preferred_element_type=jnp.float32)
    hg = hg + b1g_ref[...]
    g = hg[:, eh:eh + e]
    m = jnp.max(g, axis=-1, keepdims=True)
    p = 1.0 / jnp.sum(jnp.exp(g - m), axis=-1, keepdims=True)
    h = hg[:, :eh]
    h = 0.5 * h * (1.0 + jax.lax.erf(h * 0.7071067811865476))
    out = jnp.dot(h.astype(jnp.bfloat16), w2_ref[...],
                  preferred_element_type=jnp.float32)
    out = out + jnp.sum(b2_ref[...], axis=0, keepdims=True)
    o_ref[...] = out * p


def kernel(x, Wg, bg, W1, b1, W2, b2):
    B, S, D = x.shape
    E, _, H = W1.shape
    EH = E * H
    M = B * S
    TM = 1024

    PAD = 128
    xf = x.reshape(M, D)
    W1c = W1.transpose(1, 0, 2).reshape(D, EH)
    Wgp = jnp.pad(Wg, ((0, 0), (0, PAD - E)))
    W1g = jnp.concatenate([W1c, Wgp], axis=1).astype(jnp.bfloat16)
    b1g = jnp.pad(jnp.concatenate([b1.reshape(EH), bg]), (0, PAD - E))
    b1g = b1g.reshape(1, EH + PAD)
    W2c = W2.reshape(EH, D).astype(jnp.bfloat16)

    out = pl.pallas_call(
        _moe_kernel,
        grid=(M // TM,),
        in_specs=[
            pl.BlockSpec((TM, D), lambda i: (i, 0)),
            pl.BlockSpec((D, EH + PAD), lambda i: (0, 0)),
            pl.BlockSpec((1, EH + PAD), lambda i: (0, 0)),
            pl.BlockSpec((EH, D), lambda i: (0, 0)),
            pl.BlockSpec((E, D), lambda i: (0, 0)),
        ],
        out_specs=pl.BlockSpec((TM, D), lambda i: (i, 0)),
        out_shape=jax.ShapeDtypeStruct((M, D), jnp.float32),
    )(xf, W1g, b1g, W2c, b2)
    return out.reshape(B, S, D)


# speedup vs baseline: 1.0633x; 1.0000x over previous
import jax
import jax.numpy as jnp
from jax.experimental import pallas as pl


def _moe_kernel(x_ref, w1g_ref, b1g_ref, w2_ref, b2_ref, o_ref):
    eh = w2_ref.shape[0]
    e = b2_ref.shape[0]
    tm = x_ref.shape[0]
    b2s = jnp.sum(b2_ref[...], axis=0, keepdims=True)
    for half in range(2):
        rows = pl.ds(half * (tm // 2), tm // 2)
        xb16 = x_ref[rows, :].astype(jnp.bfloat16)
        hg = jnp.dot(xb16, w1g_ref[...], preferred_element_type=jnp.float32)
        hg = hg + b1g_ref[...]
        g = hg[:, eh:eh + e]
        m = jnp.max(g, axis=-1, keepdims=True)
        p = 1.0 / jnp.sum(jnp.exp(g - m), axis=-1, keepdims=True)
        h = hg[:, :eh]
        h = h * (1.0 + jax.lax.erf(h * 0.7071067811865476))
        out = jnp.dot(h.astype(jnp.bfloat16), w2_ref[...],
                      preferred_element_type=jnp.float32)
        o_ref[rows, :] = (out + b2s) * p


def kernel(x, Wg, bg, W1, b1, W2, b2):
    B, S, D = x.shape
    E, _, H = W1.shape
    EH = E * H
    M = B * S
    TM = 1024

    PAD = 128
    xf = x.reshape(M, D)
    W1c = W1.transpose(1, 0, 2).reshape(D, EH)
    Wgp = jnp.pad(Wg, ((0, 0), (0, PAD - E)))
    W1g = jnp.concatenate([W1c, Wgp], axis=1).astype(jnp.bfloat16)
    b1g = jnp.pad(jnp.concatenate([b1.reshape(EH), bg]), (0, PAD - E))
    b1g = b1g.reshape(1, EH + PAD)
    W2c = (0.5 * W2.reshape(EH, D)).astype(jnp.bfloat16)

    out = pl.pallas_call(
        _moe_kernel,
        grid=(M // TM,),
        in_specs=[
            pl.BlockSpec((TM, D), lambda i: (i, 0)),
            pl.BlockSpec((D, EH + PAD), lambda i: (0, 0)),
            pl.BlockSpec((1, EH + PAD), lambda i: (0, 0)),
            pl.BlockSpec((EH, D), lambda i: (0, 0)),
            pl.BlockSpec((E, D), lambda i: (0, 0)),
        ],
        out_specs=pl.BlockSpec((TM, D), lambda i: (i, 0)),
        out_shape=jax.ShapeDtypeStruct((M, D), jnp.float32),
    )(xf, W1g, b1g, W2c, b2)
    return out.reshape(B, S, D)


# speedup vs baseline: 1.1389x; 1.0711x over previous
import jax
import jax.numpy as jnp
from jax.experimental import pallas as pl
from jax.experimental.pallas import tpu as pltpu

_GPAD = 128


def _moe_kernel(x_ref, w1_ref, wg_ref, b1_ref, bg_ref, w2_ref, b2_ref,
                o_ref, w1g_s, w2_s):
    e, d, hh = w1_ref.shape
    eh = e * hh
    tm = x_ref.shape[0]

    @pl.when(pl.program_id(0) == 0)
    def _prep():
        for i in range(e):
            w1g_s[:, i * hh:(i + 1) * hh] = w1_ref[i].astype(jnp.bfloat16)
        w1g_s[:, eh:] = jnp.zeros((d, _GPAD), jnp.bfloat16)
        w1g_s[:, eh:eh + e] = wg_ref[...].astype(jnp.bfloat16)
        w2_s[...] = (w2_ref[...] * 0.5).astype(jnp.bfloat16)

    b2s = jnp.sum(b2_ref[...], axis=0, keepdims=True)
    for half in range(2):
        rows = pl.ds(half * (tm // 2), tm // 2)
        xb16 = x_ref[rows, :].astype(jnp.bfloat16)
        hg = jnp.dot(xb16, w1g_s[...], preferred_element_type=jnp.float32)
        g = hg[:, eh:eh + e] + bg_ref[...]
        m = jnp.max(g, axis=-1, keepdims=True)
        p = 1.0 / jnp.sum(jnp.exp(g - m), axis=-1, keepdims=True)
        h = hg[:, :eh] + b1_ref[...]
        h = h * (1.0 + jax.lax.erf(h * 0.7071067811865476))
        out = jnp.dot(h.astype(jnp.bfloat16), w2_s[...],
                      preferred_element_type=jnp.float32)
        o_ref[rows, :] = (out + b2s) * p


def kernel(x, Wg, bg, W1, b1, W2, b2):
    B, S, D = x.shape
    E, _, H = W1.shape
    EH = E * H
    M = B * S
    TM = 1024

    xf = x.reshape(M, D)
    W2r = W2.reshape(EH, D)
    b1r = b1.reshape(1, EH)
    bgr = bg.reshape(1, E)

    out = pl.pallas_call(
        _moe_kernel,
        grid=(M // TM,),
        in_specs=[
            pl.BlockSpec((TM, D), lambda i: (i, 0)),
            pl.BlockSpec((E, D, H), lambda i: (0, 0, 0)),
            pl.BlockSpec((D, E), lambda i: (0, 0)),
            pl.BlockSpec((1, EH), lambda i: (0, 0)),
            pl.BlockSpec((1, E), lambda i: (0, 0)),
            pl.BlockSpec((EH, D), lambda i: (0, 0)),
            pl.BlockSpec((E, D), lambda i: (0, 0)),
        ],
        out_specs=pl.BlockSpec((TM, D), lambda i: (i, 0)),
        out_shape=jax.ShapeDtypeStruct((M, D), jnp.float32),
        scratch_shapes=[
            pltpu.VMEM((D, EH + _GPAD), jnp.bfloat16),
            pltpu.VMEM((EH, D), jnp.bfloat16),
        ],
    )(xf, W1, Wg, b1r, bgr, W2r, b2)
    return out.reshape(B, S, D)
